# baseline (speedup 1.0000x reference)
import jax
import jax.numpy as jnp
from jax import lax
from jax.experimental import pallas as pl
from jax.experimental.pallas import tpu as pltpu
from jax.experimental.pallas import tpu_sc as plsc

N = 10000
E = 320000
D = 128
DH = D // 2
K = 3
INIT_W = 0.9
EPS = 1e-12

NC = 2
NS = 16
NW = NC * NS
CHUNK = 128
ROWS_W = 80
ROWS_S = 160
EPT = ROWS_W * CHUNK
E_PAD = NW * EPT
N_PAD = 10112
ROWS_T = N_PAD // NS

_mesh = plsc.VectorSubcoreMesh(core_axis_name="c", subcore_axis_name="s")


HW = 16
HSLICE = 2 * N_PAD // NS


def _deg_body(srcr_hbm, dstr_hbm, zh_hbm, ones_hbm, out_hbm,
              srcv, dstv, ones_v, stage_v, shared):
    cid = lax.axis_index("c")
    sid = lax.axis_index("s")
    wid = cid * NS + sid

    pltpu.sync_copy(srcr_hbm.at[wid], srcv)
    pltpu.sync_copy(dstr_hbm.at[wid], dstv)
    pltpu.sync_copy(ones_hbm, ones_v)
    pltpu.sync_copy(zh_hbm, stage_v)
    base = sid * HSLICE
    pltpu.sync_copy(stage_v, shared.at[pl.ds(base, HSLICE)])
    plsc.subcore_barrier()

    def _step(j, _):
        pltpu.sync_copy(ones_v, shared.at[srcv.at[j]], add=True)
        pltpu.sync_copy(ones_v, shared.at[dstv.at[j]], add=True)
        return 0

    lax.fori_loop(0, ROWS_W, _step, 0)
    plsc.subcore_barrier()

    pltpu.sync_copy(shared.at[pl.ds(base, HSLICE)], stage_v)
    pltpu.sync_copy(stage_v, out_hbm.at[cid, pl.ds(base, HSLICE)])


_deg_call = pl.kernel(
    _deg_body,
    out_type=jax.ShapeDtypeStruct((NC, 2 * N_PAD, HW), jnp.float32),
    mesh=_mesh,
    scratch_types=[
        pltpu.VMEM((ROWS_W, CHUNK), jnp.int32),
        pltpu.VMEM((ROWS_W, CHUNK), jnp.int32),
        pltpu.VMEM((CHUNK, HW), jnp.float32),
        pltpu.VMEM((HSLICE, HW), jnp.float32),
        pltpu.VMEM_SHARED((2 * N_PAD, HW), jnp.float32),
    ],
    compiler_params=pltpu.CompilerParams(use_tc_tiling_on_sc=False),
)


_sc_params = pltpu.CompilerParams(use_tc_tiling_on_sc=False)


def _spmm_body(h0_hbm, h1_hbm, srcr_hbm, dstr_hbm, zrows_hbm, out_hbm,
               srcv, dstv, rows0, rows1, acc_sh, sem):
    cid = lax.axis_index("c")
    sid = lax.axis_index("s")

    pltpu.sync_copy(srcr_hbm.at[sid], srcv)
    pltpu.sync_copy(dstr_hbm.at[sid], dstv)
    tb = sid * ROWS_T
    tail = ROWS_T - 4 * CHUNK

    pltpu.sync_copy(zrows_hbm, rows0)
    for q in range(4):
        pltpu.sync_copy(rows0, acc_sh.at[pl.ds(tb + q * CHUNK, CHUNK)])
    pltpu.sync_copy(rows0.at[pl.ds(0, tail)],
                    acc_sh.at[pl.ds(tb + 4 * CHUNK, tail)])
    plsc.subcore_barrier()

    def _make_loop(h_hbm):
        def _drain(buf):
            pltpu.make_async_copy(h_hbm.at[pl.ds(0, CHUNK)], buf, sem).wait()

        def _pair(p, _):
            j = p * 2
            _drain(rows0)
            pltpu.async_copy(h_hbm.at[srcv.at[j + 1]], rows1, sem)
            pltpu.sync_copy(rows0, acc_sh.at[dstv.at[j]], add=True)
            _drain(rows1)

            @pl.when(j + 2 < ROWS_S)
            def _next():
                pltpu.async_copy(h_hbm.at[srcv.at[j + 2]], rows0, sem)

            pltpu.sync_copy(rows1, acc_sh.at[dstv.at[j + 1]], add=True)
            return 0

        pltpu.async_copy(h_hbm.at[srcv.at[0]], rows0, sem)
        lax.fori_loop(0, ROWS_S // 2, _pair, 0)

    @pl.when(cid == 0)
    def _loop0():
        _make_loop(h0_hbm)

    @pl.when(cid == 1)
    def _loop1():
        _make_loop(h1_hbm)

    plsc.subcore_barrier()

    for q in range(4):
        buf = rows0 if q % 2 == 0 else rows1
        pltpu.sync_copy(acc_sh.at[pl.ds(tb + q * CHUNK, CHUNK)], buf)
        pltpu.sync_copy(buf, out_hbm.at[cid, pl.ds(tb + q * CHUNK, CHUNK)])
    pltpu.sync_copy(acc_sh.at[pl.ds(tb + 4 * CHUNK, tail)],
                    rows0.at[pl.ds(0, tail)])
    pltpu.sync_copy(rows0.at[pl.ds(0, tail)],
                    out_hbm.at[cid, pl.ds(tb + 4 * CHUNK, tail)])


_spmm_call = pl.kernel(
    _spmm_body,
    out_type=jax.ShapeDtypeStruct((NC, N_PAD, DH), jnp.float32),
    mesh=_mesh,
    scratch_types=[
        pltpu.VMEM((ROWS_S, CHUNK), jnp.int32),
        pltpu.VMEM((ROWS_S, CHUNK), jnp.int32),
        pltpu.VMEM((CHUNK, DH), jnp.float32),
        pltpu.VMEM((CHUNK, DH), jnp.float32),
        pltpu.VMEM_SHARED((N_PAD, DH), jnp.float32),
        pltpu.SemaphoreType.DMA,
    ],
    compiler_params=_sc_params,
)


def _norm_cols(deg_blk):
    deg_s = deg_blk[0, 0, 0, :, 0:1] + deg_blk[1, 0, 0, :, 0:1]
    deg_d = deg_blk[0, 1, 0, :, 0:1] + deg_blk[1, 1, 0, :, 0:1]
    ns = lax.rsqrt(jnp.maximum(deg_s, 1.0))
    nd = lax.rsqrt(jnp.maximum(deg_d, 1.0))
    return ns, nd


def _split(h, h_ref):
    h_ref[0] = h[:, :DH]
    h_ref[1] = h[:, DH:]


def _prep_body(deg_ref, feat_ref, h_ref):
    ns_col, _ = _norm_cols(deg_ref[...])
    _split(feat_ref[...] * ns_col, h_ref)


def _combine_body(deg_ref, p_ref, feat_ref, x_ref, h_ref):
    ns_col, nd_col = _norm_cols(deg_ref[...])
    agg = jnp.concatenate([p_ref[0], p_ref[1]], axis=1)
    x = (1.0 - INIT_W) * (agg * nd_col) + INIT_W * feat_ref[...]
    x_ref[...] = x
    _split(x * ns_col, h_ref)


def _r16(v):
    return v.astype(jnp.bfloat16).astype(jnp.float32)


def _se_body(w1_ref, w2_ref, deg_ref, p_ref, feat_ref, x1_ref, x2_ref,
             out_ref):
    _, nd_col = _norm_cols(deg_ref[...])
    agg = jnp.concatenate([p_ref[0], p_ref[1]], axis=1)
    x3 = (1.0 - INIT_W) * (agg * nd_col) + INIT_W * feat_ref[...]
    xs = [feat_ref[...], x1_ref[...], x2_ref[...], x3]
    sq = [jnp.sum(x, axis=1, keepdims=True) for x in xs]
    nrm = jnp.sqrt(sq[0] * sq[0] + sq[1] * sq[1] + sq[2] * sq[2] + sq[3] * sq[3])
    den = jnp.maximum(nrm, EPS)
    s = [_r16(q / den) for q in sq]
    e = []
    for c in range(4):
        acc = s[0] * _r16(w1_ref[0, c])
        for k in range(1, 4):
            acc = acc + s[k] * _r16(w1_ref[k, c])
        e.append(_r16(jnp.clip(acc, 0.0, 6.0)))
    g = []
    for k in range(4):
        acc = e[0] * _r16(w2_ref[0, k])
        for c in range(1, 4):
            acc = acc + e[c] * _r16(w2_ref[c, k])
        g.append(acc)
    nrm2 = jnp.sqrt(g[0] * g[0] + g[1] * g[1] + g[2] * g[2] + g[3] * g[3])
    den2 = jnp.maximum(nrm2, EPS)
    out_ref[...] = (
        xs[0] * (g[0] / den2)
        + xs[1] * (g[1] / den2)
        + xs[2] * (g[2] / den2)
        + xs[3] * (g[3] / den2)
    )


_BLK = CHUNK
_xspec = pl.BlockSpec((_BLK, D), lambda i: (i, 0))
_hspec = pl.BlockSpec((NC, _BLK, DH), lambda i: (0, i, 0))
_dspec = pl.BlockSpec((NC, 2, 1, CHUNK, HW), lambda i: (0, 0, i, 0, 0))
_wspec = pl.BlockSpec(memory_space=pltpu.SMEM)
_GRID = (N_PAD // _BLK,)


def _tc_prep(deg5, feat_pad):
    return pl.pallas_call(
        _prep_body,
        grid=_GRID,
        in_specs=[_dspec, _xspec],
        out_specs=_hspec,
        out_shape=jax.ShapeDtypeStruct((NC, N_PAD, DH), jnp.float32),
    )(deg5, feat_pad)


def _tc_combine(deg5, partial, feat_pad):
    return pl.pallas_call(
        _combine_body,
        grid=_GRID,
        in_specs=[_dspec, _hspec, _xspec],
        out_specs=[_xspec, _hspec],
        out_shape=[jax.ShapeDtypeStruct((N_PAD, D), jnp.float32),
                   jax.ShapeDtypeStruct((NC, N_PAD, DH), jnp.float32)],
    )(deg5, partial, feat_pad)


def _tc_se(w1, w2, deg5, partial, feat_pad, x1, x2):
    return pl.pallas_call(
        _se_body,
        grid=_GRID,
        in_specs=[_wspec, _wspec, _dspec, _hspec, _xspec, _xspec, _xspec],
        out_specs=_xspec,
        out_shape=jax.ShapeDtypeStruct((N_PAD, D), jnp.float32),
    )(w1, w2, deg5, partial, feat_pad, x1, x2)


def kernel(n_feat, edge_index, e_weight1, e_weight2):
    src = edge_index[0]
    dst = edge_index[1]
    pad = jnp.full((E_PAD - E,), N, jnp.int32)
    src_p = jnp.concatenate([src, pad])
    dst_p = jnp.concatenate([dst, pad])
    src_rows32 = src_p.reshape(NW, ROWS_W, CHUNK)
    dst_deg_rows32 = (dst_p + N_PAD).reshape(NW, ROWS_W, CHUNK)
    src_rows16 = src_p.reshape(NS, ROWS_S, CHUNK)
    dst_rows16 = dst_p.reshape(NS, ROWS_S, CHUNK)
    feat_pad = jnp.concatenate(
        [n_feat, jnp.zeros((N_PAD - N, D), jnp.float32)])
    zrows = jnp.zeros((CHUNK, DH), jnp.float32)
    zh = jnp.zeros((HSLICE, HW), jnp.float32)
    ones_rows = jnp.ones((CHUNK, HW), jnp.float32)

    deg = _deg_call(src_rows32, dst_deg_rows32, zh, ones_rows)
    deg5 = deg.reshape(NC, 2, N_PAD // CHUNK, CHUNK, HW)

    h = _tc_prep(deg5, feat_pad)
    xs = [feat_pad]
    for _ in range(K - 1):
        partial = _spmm_call(h[0], h[1], src_rows16, dst_rows16, zrows)
        x_next, h = _tc_combine(deg5, partial, feat_pad)
        xs.append(x_next)
    partial = _spmm_call(h[0], h[1], src_rows16, dst_rows16, zrows)
    out_pad = _tc_se(e_weight1, e_weight2, deg5, partial, feat_pad,
                     xs[1], xs[2])
    return out_pad[:N]


# speedup vs baseline: 1.2484x; 1.2484x over previous
import jax
import jax.numpy as jnp
from jax import lax
from jax.experimental import pallas as pl
from jax.experimental.pallas import tpu as pltpu
from jax.experimental.pallas import tpu_sc as plsc

N = 10000
E = 320000
D = 128
DH = D // 2
K = 3
INIT_W = 0.9
EPS = 1e-12

NC = 2
NS = 16
NW = NC * NS
CHUNK = 128
ROWS_W = 79
ROWS_S = 158
EPT = ROWS_W * CHUNK
E_PAD = NW * EPT
N_PAD = 10112
ROWS_T = N_PAD // NS

_mesh = plsc.VectorSubcoreMesh(core_axis_name="c", subcore_axis_name="s")


HW = 16
HSLICE = 2 * N_PAD // NS


def _deg_body(srcr_hbm, dstr_hbm, zh_hbm, ones_hbm, out_hbm,
              srcv, dstv, ones_v, stage_v, shared):
    cid = lax.axis_index("c")
    sid = lax.axis_index("s")
    wid = cid * NS + sid

    pltpu.sync_copy(srcr_hbm.at[wid], srcv)
    pltpu.sync_copy(dstr_hbm.at[wid], dstv)
    pltpu.sync_copy(ones_hbm, ones_v)
    pltpu.sync_copy(zh_hbm, stage_v)
    base = sid * HSLICE
    pltpu.sync_copy(stage_v, shared.at[pl.ds(base, HSLICE)])
    plsc.subcore_barrier()

    def _step(j, _):
        pltpu.sync_copy(ones_v, shared.at[srcv.at[j]], add=True)
        pltpu.sync_copy(ones_v, shared.at[dstv.at[j]], add=True)
        return 0

    lax.fori_loop(0, ROWS_W, _step, 0)
    plsc.subcore_barrier()

    pltpu.sync_copy(shared.at[pl.ds(base, HSLICE)], stage_v)
    pltpu.sync_copy(stage_v, out_hbm.at[cid, pl.ds(base, HSLICE)])


_deg_call = pl.kernel(
    _deg_body,
    out_type=jax.ShapeDtypeStruct((NC, 2 * N_PAD, HW), jnp.float32),
    mesh=_mesh,
    scratch_types=[
        pltpu.VMEM((ROWS_W, CHUNK), jnp.int32),
        pltpu.VMEM((ROWS_W, CHUNK), jnp.int32),
        pltpu.VMEM((CHUNK, HW), jnp.float32),
        pltpu.VMEM((HSLICE, HW), jnp.float32),
        pltpu.VMEM_SHARED((2 * N_PAD, HW), jnp.float32),
    ],
    compiler_params=pltpu.CompilerParams(use_tc_tiling_on_sc=False),
)


_sc_params = pltpu.CompilerParams(use_tc_tiling_on_sc=False)


def _spmm_body(h0_hbm, h1_hbm, srcr_hbm, dstr_hbm, zrows_hbm, out_hbm,
               srcv, dstv, rows0, rows1, acc_sh, sem):
    cid = lax.axis_index("c")
    sid = lax.axis_index("s")

    pltpu.sync_copy(srcr_hbm.at[sid], srcv)
    pltpu.sync_copy(dstr_hbm.at[sid], dstv)
    tb = sid * ROWS_T
    tail = ROWS_T - 4 * CHUNK

    pltpu.sync_copy(zrows_hbm, rows0)
    for q in range(4):
        pltpu.sync_copy(rows0, acc_sh.at[pl.ds(tb + q * CHUNK, CHUNK)])
    pltpu.sync_copy(rows0.at[pl.ds(0, tail)],
                    acc_sh.at[pl.ds(tb + 4 * CHUNK, tail)])
    plsc.subcore_barrier()

    def _make_loop(h_hbm):
        def _drain(buf):
            pltpu.make_async_copy(h_hbm.at[pl.ds(0, CHUNK)], buf, sem).wait()

        def _pair(p, _):
            j = p * 2
            _drain(rows0)
            pltpu.async_copy(h_hbm.at[srcv.at[j + 1]], rows1, sem)
            pltpu.sync_copy(rows0, acc_sh.at[dstv.at[j]], add=True)
            _drain(rows1)

            @pl.when(j + 2 < ROWS_S)
            def _next():
                pltpu.async_copy(h_hbm.at[srcv.at[j + 2]], rows0, sem)

            pltpu.sync_copy(rows1, acc_sh.at[dstv.at[j + 1]], add=True)
            return 0

        pltpu.async_copy(h_hbm.at[srcv.at[0]], rows0, sem)
        lax.fori_loop(0, ROWS_S // 2, _pair, 0)

    @pl.when(cid == 0)
    def _loop0():
        _make_loop(h0_hbm)

    @pl.when(cid == 1)
    def _loop1():
        _make_loop(h1_hbm)

    plsc.subcore_barrier()

    for q in range(4):
        buf = rows0 if q % 2 == 0 else rows1
        pltpu.sync_copy(acc_sh.at[pl.ds(tb + q * CHUNK, CHUNK)], buf)
        pltpu.sync_copy(buf, out_hbm.at[cid, pl.ds(tb + q * CHUNK, CHUNK)])
    pltpu.sync_copy(acc_sh.at[pl.ds(tb + 4 * CHUNK, tail)],
                    rows0.at[pl.ds(0, tail)])
    pltpu.sync_copy(rows0.at[pl.ds(0, tail)],
                    out_hbm.at[cid, pl.ds(tb + 4 * CHUNK, tail)])


_spmm_call = pl.kernel(
    _spmm_body,
    out_type=jax.ShapeDtypeStruct((NC, N_PAD, DH), jnp.float32),
    mesh=_mesh,
    scratch_types=[
        pltpu.VMEM((ROWS_S, CHUNK), jnp.int32),
        pltpu.VMEM((ROWS_S, CHUNK), jnp.int32),
        pltpu.VMEM((CHUNK, DH), jnp.float32),
        pltpu.VMEM((CHUNK, DH), jnp.float32),
        pltpu.VMEM_SHARED((N_PAD, DH), jnp.float32),
        pltpu.SemaphoreType.DMA,
    ],
    compiler_params=_sc_params,
)


def _norm_cols(deg_blk):
    deg_s = deg_blk[0, 0, 0, :, 0:1] + deg_blk[1, 0, 0, :, 0:1]
    deg_d = deg_blk[0, 1, 0, :, 0:1] + deg_blk[1, 1, 0, :, 0:1]
    ns = lax.rsqrt(jnp.maximum(deg_s, 1.0))
    nd = lax.rsqrt(jnp.maximum(deg_d, 1.0))
    return ns, nd


def _split(h, h_ref):
    h_ref[0] = h[:, :DH]
    h_ref[1] = h[:, DH:]


def _prep_body(deg_ref, feat_ref, h_ref):
    ns_col, _ = _norm_cols(deg_ref[...])
    _split(feat_ref[...] * ns_col, h_ref)


def _combine_body(deg_ref, p_ref, feat_ref, x_ref, h_ref):
    ns_col, nd_col = _norm_cols(deg_ref[...])
    agg = jnp.concatenate([p_ref[0], p_ref[1]], axis=1)
    x = (1.0 - INIT_W) * (agg * nd_col) + INIT_W * feat_ref[...]
    x_ref[...] = x
    _split(x * ns_col, h_ref)


def _r16(v):
    return v.astype(jnp.bfloat16).astype(jnp.float32)


def _se_body(w1_ref, w2_ref, deg_ref, p_ref, feat_ref, x1_ref, x2_ref,
             out_ref):
    _, nd_col = _norm_cols(deg_ref[...])
    agg = jnp.concatenate([p_ref[0], p_ref[1]], axis=1)
    x3 = (1.0 - INIT_W) * (agg * nd_col) + INIT_W * feat_ref[...]
    xs = [feat_ref[...], x1_ref[...], x2_ref[...], x3]
    sq = [jnp.sum(x, axis=1, keepdims=True) for x in xs]
    nrm = jnp.sqrt(sq[0] * sq[0] + sq[1] * sq[1] + sq[2] * sq[2] + sq[3] * sq[3])
    den = jnp.maximum(nrm, EPS)
    s = [_r16(q / den) for q in sq]
    e = []
    for c in range(4):
        acc = s[0] * _r16(w1_ref[0, c])
        for k in range(1, 4):
            acc = acc + s[k] * _r16(w1_ref[k, c])
        e.append(_r16(jnp.clip(acc, 0.0, 6.0)))
    g = []
    for k in range(4):
        acc = e[0] * _r16(w2_ref[0, k])
        for c in range(1, 4):
            acc = acc + e[c] * _r16(w2_ref[c, k])
        g.append(acc)
    nrm2 = jnp.sqrt(g[0] * g[0] + g[1] * g[1] + g[2] * g[2] + g[3] * g[3])
    den2 = jnp.maximum(nrm2, EPS)
    out_ref[...] = (
        xs[0] * (g[0] / den2)
        + xs[1] * (g[1] / den2)
        + xs[2] * (g[2] / den2)
        + xs[3] * (g[3] / den2)
    )


_BLK = CHUNK
_xspec = pl.BlockSpec((_BLK, D), lambda i: (i, 0))
_hspec = pl.BlockSpec((NC, _BLK, DH), lambda i: (0, i, 0))
_dspec = pl.BlockSpec((NC, 2, 1, CHUNK, HW), lambda i: (0, 0, i, 0, 0))
_wspec = pl.BlockSpec(memory_space=pltpu.SMEM)
_GRID = (N_PAD // _BLK,)


def _tc_prep(deg5, feat_pad):
    return pl.pallas_call(
        _prep_body,
        grid=_GRID,
        in_specs=[_dspec, _xspec],
        out_specs=_hspec,
        out_shape=jax.ShapeDtypeStruct((NC, N_PAD, DH), jnp.float32),
    )(deg5, feat_pad)


def _tc_combine(deg5, partial, feat_pad):
    return pl.pallas_call(
        _combine_body,
        grid=_GRID,
        in_specs=[_dspec, _hspec, _xspec],
        out_specs=[_xspec, _hspec],
        out_shape=[jax.ShapeDtypeStruct((N_PAD, D), jnp.float32),
                   jax.ShapeDtypeStruct((NC, N_PAD, DH), jnp.float32)],
    )(deg5, partial, feat_pad)


def _tc_se(w1, w2, deg5, partial, feat_pad, x1, x2):
    return pl.pallas_call(
        _se_body,
        grid=_GRID,
        in_specs=[_wspec, _wspec, _dspec, _hspec, _xspec, _xspec, _xspec],
        out_specs=_xspec,
        out_shape=jax.ShapeDtypeStruct((N_PAD, D), jnp.float32),
    )(w1, w2, deg5, partial, feat_pad, x1, x2)


def kernel(n_feat, edge_index, e_weight1, e_weight2):
    src = edge_index[0]
    dst = edge_index[1]
    pad = jnp.full((E_PAD - E,), N, jnp.int32)
    src_p = jnp.concatenate([src, pad])
    dst_p = jnp.concatenate([dst, pad])
    src_rows32 = src_p.reshape(NW, ROWS_W, CHUNK)
    dst_deg_rows32 = (dst_p + N_PAD).reshape(NW, ROWS_W, CHUNK)
    src_rows16 = src_p.reshape(NS, ROWS_S, CHUNK)
    dst_rows16 = dst_p.reshape(NS, ROWS_S, CHUNK)
    feat_pad = jnp.concatenate(
        [n_feat, jnp.zeros((N_PAD - N, D), jnp.float32)])
    zrows = jnp.zeros((CHUNK, DH), jnp.float32)
    zh = jnp.zeros((HSLICE, HW), jnp.float32)
    ones_rows = jnp.ones((CHUNK, HW), jnp.float32)

    deg = _deg_call(src_rows32, dst_deg_rows32, zh, ones_rows)
    deg5 = deg.reshape(NC, 2, N_PAD // CHUNK, CHUNK, HW)

    h = _tc_prep(deg5, feat_pad)
    xs = [feat_pad]
    for _ in range(K - 1):
        partial = _spmm_call(h[0], h[1], src_rows16, dst_rows16, zrows)
        x_next, h = _tc_combine(deg5, partial, feat_pad)
        xs.append(x_next)
    partial = _spmm_call(h[0], h[1], src_rows16, dst_rows16, zrows)
    out_pad = _tc_se(e_weight1, e_weight2, deg5, partial, feat_pad,
                     xs[1], xs[2])
    return out_pad[:N]


# speedup vs baseline: 1.5628x; 1.2519x over previous
import jax
import jax.numpy as jnp
from jax import lax
from jax.experimental import pallas as pl
from jax.experimental.pallas import tpu as pltpu
from jax.experimental.pallas import tpu_sc as plsc

N = 10000
E = 320000
D = 128
DH = D // 2
K = 3
INIT_W = 0.9
EPS = 1e-12

NC = 2
NS = 16
NW = NC * NS
CHUNK = 128
ROWS_W = 79
ROWS_S = 158
EPT = ROWS_W * CHUNK
E_PAD = NW * EPT
N_PAD = 10112
ROWS_T = N_PAD // NS

_mesh = plsc.VectorSubcoreMesh(core_axis_name="c", subcore_axis_name="s")


HW = 16
HSLICE = 2 * N_PAD // NS


def _deg_body(srcr_hbm, dstr_hbm, zh_hbm, ones_hbm, out_hbm,
              srcv, dstv, ones_v, stage_v, shared):
    cid = lax.axis_index("c")
    sid = lax.axis_index("s")
    wid = cid * NS + sid

    pltpu.sync_copy(srcr_hbm.at[wid], srcv)
    pltpu.sync_copy(dstr_hbm.at[wid], dstv)
    pltpu.sync_copy(ones_hbm, ones_v)
    pltpu.sync_copy(zh_hbm, stage_v)
    base = sid * HSLICE
    pltpu.sync_copy(stage_v, shared.at[pl.ds(base, HSLICE)])
    plsc.subcore_barrier()

    def _step(j, _):
        pltpu.sync_copy(ones_v, shared.at[srcv.at[j]], add=True)
        pltpu.sync_copy(ones_v, shared.at[dstv.at[j]], add=True)
        return 0

    lax.fori_loop(0, ROWS_W, _step, 0)
    plsc.subcore_barrier()

    pltpu.sync_copy(shared.at[pl.ds(base, HSLICE)], stage_v)
    pltpu.sync_copy(stage_v, out_hbm.at[cid, pl.ds(base, HSLICE)])


_deg_call = pl.kernel(
    _deg_body,
    out_type=jax.ShapeDtypeStruct((NC, 2 * N_PAD, HW), jnp.float32),
    mesh=_mesh,
    scratch_types=[
        pltpu.VMEM((ROWS_W, CHUNK), jnp.int32),
        pltpu.VMEM((ROWS_W, CHUNK), jnp.int32),
        pltpu.VMEM((CHUNK, HW), jnp.float32),
        pltpu.VMEM((HSLICE, HW), jnp.float32),
        pltpu.VMEM_SHARED((2 * N_PAD, HW), jnp.float32),
    ],
    compiler_params=pltpu.CompilerParams(use_tc_tiling_on_sc=False),
)


_sc_params = pltpu.CompilerParams(use_tc_tiling_on_sc=False)


def _spmm_body(h0_hbm, h1_hbm, srcr_hbm, dstr_hbm, zrows_hbm, out_hbm,
               srcv, dstv, rows0, rows1, acc_sh, sem):
    cid = lax.axis_index("c")
    sid = lax.axis_index("s")

    pltpu.sync_copy(srcr_hbm.at[sid], srcv)
    pltpu.sync_copy(dstr_hbm.at[sid], dstv)
    tb = sid * ROWS_T
    tail = ROWS_T - 4 * CHUNK

    pltpu.sync_copy(zrows_hbm, rows0)
    for q in range(4):
        pltpu.sync_copy(rows0, acc_sh.at[pl.ds(tb + q * CHUNK, CHUNK)])
    pltpu.sync_copy(rows0.at[pl.ds(0, tail)],
                    acc_sh.at[pl.ds(tb + 4 * CHUNK, tail)])
    plsc.subcore_barrier()

    def _make_loop(h_hbm):
        def _drain(buf):
            pltpu.make_async_copy(h_hbm.at[pl.ds(0, CHUNK)], buf, sem).wait()

        def _pair(p, _):
            j = p * 2
            _drain(rows0)
            pltpu.async_copy(h_hbm.at[srcv.at[j + 1]], rows1, sem)
            pltpu.sync_copy(rows0, acc_sh.at[dstv.at[j]], add=True)
            _drain(rows1)

            @pl.when(j + 2 < ROWS_S)
            def _next():
                pltpu.async_copy(h_hbm.at[srcv.at[j + 2]], rows0, sem)

            pltpu.sync_copy(rows1, acc_sh.at[dstv.at[j + 1]], add=True)
            return 0

        pltpu.async_copy(h_hbm.at[srcv.at[0]], rows0, sem)
        lax.fori_loop(0, ROWS_S // 2, _pair, 0)

    @pl.when(cid == 0)
    def _loop0():
        _make_loop(h0_hbm)

    @pl.when(cid == 1)
    def _loop1():
        _make_loop(h1_hbm)

    plsc.subcore_barrier()

    for q in range(4):
        buf = rows0 if q % 2 == 0 else rows1
        pltpu.sync_copy(acc_sh.at[pl.ds(tb + q * CHUNK, CHUNK)], buf)
        pltpu.sync_copy(buf, out_hbm.at[cid, pl.ds(tb + q * CHUNK, CHUNK)])
    pltpu.sync_copy(acc_sh.at[pl.ds(tb + 4 * CHUNK, tail)],
                    rows0.at[pl.ds(0, tail)])
    pltpu.sync_copy(rows0.at[pl.ds(0, tail)],
                    out_hbm.at[cid, pl.ds(tb + 4 * CHUNK, tail)])


_spmm_call = pl.kernel(
    _spmm_body,
    out_type=jax.ShapeDtypeStruct((NC, N_PAD, DH), jnp.float32),
    mesh=_mesh,
    scratch_types=[
        pltpu.VMEM((ROWS_S, CHUNK), jnp.int32),
        pltpu.VMEM((ROWS_S, CHUNK), jnp.int32),
        pltpu.VMEM((CHUNK, DH), jnp.float32),
        pltpu.VMEM((CHUNK, DH), jnp.float32),
        pltpu.VMEM_SHARED((N_PAD, DH), jnp.float32),
        pltpu.SemaphoreType.DMA,
    ],
    compiler_params=_sc_params,
)


def _norm_cols(deg_blk):
    deg_s = deg_blk[0, 0, 0, :, 0:1] + deg_blk[1, 0, 0, :, 0:1]
    deg_d = deg_blk[0, 1, 0, :, 0:1] + deg_blk[1, 1, 0, :, 0:1]
    ns = lax.rsqrt(jnp.maximum(deg_s, 1.0))
    nd = lax.rsqrt(jnp.maximum(deg_d, 1.0))
    return ns, nd


def _split(h, h_ref):
    h_ref[0] = h[:, :DH]
    h_ref[1] = h[:, DH:]


def _prep_body(deg_ref, feat_ref, h_ref):
    ns_col, _ = _norm_cols(deg_ref[...])
    _split(feat_ref[...] * ns_col, h_ref)


def _combine_body(deg_ref, p_ref, feat_ref, x_ref, h_ref):
    ns_col, nd_col = _norm_cols(deg_ref[...])
    agg = jnp.concatenate([p_ref[0], p_ref[1]], axis=1)
    x = (1.0 - INIT_W) * (agg * nd_col) + INIT_W * feat_ref[...]
    x_ref[...] = x
    _split(x * ns_col, h_ref)


def _r16(v):
    return v.astype(jnp.bfloat16).astype(jnp.float32)


def _se_body(w1_ref, w2_ref, deg_ref, p_ref, feat_ref, x1_ref, x2_ref,
             out_ref):
    _, nd_col = _norm_cols(deg_ref[...])
    agg = jnp.concatenate([p_ref[0], p_ref[1]], axis=1)
    x3 = (1.0 - INIT_W) * (agg * nd_col) + INIT_W * feat_ref[...]
    xs = [feat_ref[...], x1_ref[...], x2_ref[...], x3]
    sq = [jnp.sum(x, axis=1, keepdims=True) for x in xs]
    nrm = jnp.sqrt(sq[0] * sq[0] + sq[1] * sq[1] + sq[2] * sq[2] + sq[3] * sq[3])
    den = jnp.maximum(nrm, EPS)
    s = [_r16(q / den) for q in sq]
    e = []
    for c in range(4):
        acc = s[0] * _r16(w1_ref[0, c])
        for k in range(1, 4):
            acc = acc + s[k] * _r16(w1_ref[k, c])
        e.append(_r16(jnp.clip(acc, 0.0, 6.0)))
    g = []
    for k in range(4):
        acc = e[0] * _r16(w2_ref[0, k])
        for c in range(1, 4):
            acc = acc + e[c] * _r16(w2_ref[c, k])
        g.append(acc)
    nrm2 = jnp.sqrt(g[0] * g[0] + g[1] * g[1] + g[2] * g[2] + g[3] * g[3])
    den2 = jnp.maximum(nrm2, EPS)
    out_ref[...] = (
        xs[0] * (g[0] / den2)
        + xs[1] * (g[1] / den2)
        + xs[2] * (g[2] / den2)
        + xs[3] * (g[3] / den2)
    )


_BLK = CHUNK
_xspec = pl.BlockSpec((_BLK, D), lambda i: (i, 0))
_hspec = pl.BlockSpec((NC, _BLK, DH), lambda i: (0, i, 0))
_dspec = pl.BlockSpec((NC, 2, 1, CHUNK, HW), lambda i: (0, 0, i, 0, 0))
_wspec = pl.BlockSpec(memory_space=pltpu.SMEM)
_GRID = (N_PAD // _BLK,)


def _tc_prep(deg5, feat_pad):
    return pl.pallas_call(
        _prep_body,
        grid=_GRID,
        in_specs=[_dspec, _xspec],
        out_specs=_hspec,
        out_shape=jax.ShapeDtypeStruct((NC, N_PAD, DH), jnp.float32),
    )(deg5, feat_pad)


def _tc_combine(deg5, partial, feat_pad):
    return pl.pallas_call(
        _combine_body,
        grid=_GRID,
        in_specs=[_dspec, _hspec, _xspec],
        out_specs=[_xspec, _hspec],
        out_shape=[jax.ShapeDtypeStruct((N_PAD, D), jnp.float32),
                   jax.ShapeDtypeStruct((NC, N_PAD, DH), jnp.float32)],
    )(deg5, partial, feat_pad)


def _tc_se(w1, w2, deg5, partial, feat_pad, x1, x2):
    return pl.pallas_call(
        _se_body,
        grid=_GRID,
        in_specs=[_wspec, _wspec, _dspec, _hspec, _xspec, _xspec, _xspec],
        out_specs=_xspec,
        out_shape=jax.ShapeDtypeStruct((N_PAD, D), jnp.float32),
    )(w1, w2, deg5, partial, feat_pad, x1, x2)


def kernel(n_feat, edge_index, e_weight1, e_weight2):
    src = edge_index[0]
    dst = edge_index[1]
    pad = N + jnp.arange(E_PAD - E, dtype=jnp.int32) % (N_PAD - N)
    src_p = jnp.concatenate([src, pad])
    dst_p = jnp.concatenate([dst, pad])
    src_rows32 = src_p.reshape(NW, ROWS_W, CHUNK)
    dst_deg_rows32 = (dst_p + N_PAD).reshape(NW, ROWS_W, CHUNK)
    src_rows16 = src_p.reshape(NS, ROWS_S, CHUNK)
    dst_rows16 = dst_p.reshape(NS, ROWS_S, CHUNK)
    feat_pad = jnp.concatenate(
        [n_feat, jnp.zeros((N_PAD - N, D), jnp.float32)])
    zrows = jnp.zeros((CHUNK, DH), jnp.float32)
    zh = jnp.zeros((HSLICE, HW), jnp.float32)
    ones_rows = jnp.ones((CHUNK, HW), jnp.float32)

    deg = _deg_call(src_rows32, dst_deg_rows32, zh, ones_rows)
    deg5 = deg.reshape(NC, 2, N_PAD // CHUNK, CHUNK, HW)

    h = _tc_prep(deg5, feat_pad)
    xs = [feat_pad]
    for _ in range(K - 1):
        partial = _spmm_call(h[0], h[1], src_rows16, dst_rows16, zrows)
        x_next, h = _tc_combine(deg5, partial, feat_pad)
        xs.append(x_next)
    partial = _spmm_call(h[0], h[1], src_rows16, dst_rows16, zrows)
    out_pad = _tc_se(e_weight1, e_weight2, deg5, partial, feat_pad,
                     xs[1], xs[2])
    return out_pad[:N]
